# baseline (speedup 1.0000x reference)
import functools

import jax
import jax.numpy as jnp
from jax import lax
from jax.experimental import pallas as pl
from jax.experimental.pallas import tpu as pltpu
from jax.experimental.pallas import tpu_sc as plsc

BATCH = 16384
RANK_K = 32
H1 = 256
H2 = 64
NUM_WORKERS = 32
B_PER_W = BATCH // NUM_WORKERS


def _gather_body(u_tab, v_tab, i_hbm, j_hbm, u_out, v_out,
                 idx_i, idx_j, rows_u, rows_v, sem):
    wid = lax.axis_index("s") * 2 + lax.axis_index("c")
    base = wid * B_PER_W
    pltpu.sync_copy(i_hbm.at[pl.ds(base, B_PER_W)], idx_i)
    pltpu.sync_copy(j_hbm.at[pl.ds(base, B_PER_W)], idx_j)
    cu = pltpu.async_copy(u_tab.at[idx_i], rows_u, sem)
    cv = pltpu.async_copy(v_tab.at[idx_j], rows_v, sem)
    cu.wait()
    cv.wait()
    pltpu.sync_copy(rows_u, u_out.at[pl.ds(base, B_PER_W)])
    pltpu.sync_copy(rows_v, v_out.at[pl.ds(base, B_PER_W)])


def _make_gather():
    mesh = plsc.VectorSubcoreMesh(core_axis_name="c", subcore_axis_name="s")
    return pl.kernel(
        _gather_body,
        out_type=(
            jax.ShapeDtypeStruct((BATCH, RANK_K), jnp.float32),
            jax.ShapeDtypeStruct((BATCH, RANK_K), jnp.float32),
        ),
        mesh=mesh,
        scratch_types=[
            pltpu.VMEM((B_PER_W,), jnp.int32),
            pltpu.VMEM((B_PER_W,), jnp.int32),
            pltpu.VMEM((B_PER_W, RANK_K), jnp.float32),
            pltpu.VMEM((B_PER_W, RANK_K), jnp.float32),
            pltpu.SemaphoreType.DMA,
        ],
        compiler_params=pltpu.CompilerParams(use_tc_tiling_on_sc=False),
    )


def _mlp_body(u_ref, v_ref, w1u_ref, w1v_ref, w2_ref, wl_ref, out_ref):
    h = lax.dot_general(u_ref[...], w1u_ref[...], (((1,), (1,)), ((), ())),
                        preferred_element_type=jnp.float32)
    h = h + lax.dot_general(v_ref[...], w1v_ref[...], (((1,), (1,)), ((), ())),
                            preferred_element_type=jnp.float32)
    h = 0.5 * h * (1.0 + lax.erf(h * 0.7071067811865476))
    y = lax.dot_general(h, w2_ref[...], (((1,), (1,)), ((), ())),
                        preferred_element_type=jnp.float32)
    out_ref[...] = jnp.sum(y * wl_ref[...], axis=1)


def _make_mlp(bb):
    return pl.pallas_call(
        _mlp_body,
        grid=(BATCH // bb,),
        in_specs=[
            pl.BlockSpec((bb, RANK_K), lambda b: (b, 0)),
            pl.BlockSpec((bb, RANK_K), lambda b: (b, 0)),
            pl.BlockSpec((H1, RANK_K), lambda b: (0, 0)),
            pl.BlockSpec((H1, RANK_K), lambda b: (0, 0)),
            pl.BlockSpec((H2, H1), lambda b: (0, 0)),
            pl.BlockSpec((1, H2), lambda b: (0, 0)),
        ],
        out_specs=pl.BlockSpec((bb,), lambda b: (b,)),
        out_shape=jax.ShapeDtypeStruct((BATCH,), jnp.float32),
    )


def kernel(i, j, U, V, W1, W2, Wl):
    i = i.astype(jnp.int32)
    j = j.astype(jnp.int32)
    u, v = _make_gather()(U, V, i, j)
    w1u = W1[:, :RANK_K]
    w1v = W1[:, RANK_K:]
    return _make_mlp(2048)(u, v, w1u, w1v, W2, Wl)
